# baseline (speedup 1.0000x reference)
import functools

import jax
import jax.numpy as jnp
from jax import lax
from jax.experimental import pallas as pl
from jax.experimental.pallas import tpu as pltpu
from jax.experimental.pallas import tpu_sc as plsc

B, K, D, E, P = 128, 36, 2048, 1024, 5
M = B * K
TILE = 512
TILEA = 16 * K
KP = 48

_PARALLEL = pltpu.CompilerParams(dimension_semantics=("parallel",))

_NC, _NS = 2, 16
_NW = _NC * _NS
_RPW = M // _NW
_GRP = _RPW // 16


_NCHUNK = M // 128


def _select_chunk(rt_hbm, out_hbm, rv, sv, chunk):
    col0 = chunk * 128
    pltpu.sync_copy(rt_hbm.at[:, pl.ds(col0, 128)], rv)
    for g in range(8):
        lo = g * 16
        cnt = jnp.zeros((16,), jnp.float32)
        for j in range(K):
            r_j = rv[j, lo:lo + 16]
            cnt = cnt + r_j
            sv[j, lo:lo + 16] = jnp.where(
                (r_j == 1.0) & (cnt <= float(P)), 1.0, 0.0)
        sv[K, lo:lo + 16] = float(P) - jnp.minimum(cnt, float(P))
        zero = jnp.zeros((16,), jnp.float32)
        for j in range(K + 1, KP):
            sv[j, lo:lo + 16] = zero
    pltpu.sync_copy(sv, out_hbm.at[:, pl.ds(col0, 128)])


def _select_body(rt_hbm, out_hbm, rv, sv):
    wid = lax.axis_index("s") * _NC + lax.axis_index("c")
    _select_chunk(rt_hbm, out_hbm, rv, sv, wid)

    @pl.when(wid < _NCHUNK - _NW)
    def _():
        _select_chunk(rt_hbm, out_hbm, rv, sv, wid + _NW)


def _select(rt):
    mesh = plsc.VectorSubcoreMesh(core_axis_name="c", subcore_axis_name="s")
    return pl.kernel(
        _select_body,
        mesh=mesh,
        out_type=jax.ShapeDtypeStruct((KP, M), jnp.float32),
        scratch_types=[pltpu.VMEM((KP, 128), jnp.float32),
                       pltpu.VMEM((KP, 128), jnp.float32)],
    )(rt)


def _kernelA_body(x_ref, bb_ref, w1ig_ref, w1in_ref, w1eg_ref, w1en_ref,
                  gb1_ref, nb1_ref, w2n_ref, nb2_ref, gw2c_ref, gb2_ref,
                  vm_ref):
    X = x_ref[...].astype(jnp.bfloat16)
    bb = bb_ref[...]
    area = (bb[:, 2:3] - bb[:, 0:1]) * (bb[:, 3:4] - bb[:, 1:2])
    extras = (jnp.concatenate(
        [bb, area, jnp.zeros((TILE, 3), jnp.float32)], axis=1)
        * 0.1).astype(jnp.bfloat16)
    hg = jnp.dot(X, w1ig_ref[...], preferred_element_type=jnp.float32)
    hn = jnp.dot(X, w1in_ref[...], preferred_element_type=jnp.float32)
    hg = hg + jnp.dot(extras, w1eg_ref[...],
                      preferred_element_type=jnp.float32) + gb1_ref[...]
    hn = hn + jnp.dot(extras, w1en_ref[...],
                      preferred_element_type=jnp.float32) + nb1_ref[...]
    hg = jnp.maximum(hg, 0.0).astype(jnp.bfloat16)
    hn = jnp.maximum(hn, 0.0).astype(jnp.bfloat16)
    v = jnp.dot(hn, w2n_ref[...], preferred_element_type=jnp.float32)
    v = v + nb2_ref[...]
    gate = jnp.dot(hg, gw2c_ref[...],
                   preferred_element_type=jnp.float32)[:, 0:1] + gb2_ref[...]
    m = jax.nn.sigmoid(gate)
    vm_ref[...] = (m * v).astype(jnp.bfloat16)


def _kernelB_body(selp_ref, vm_ref, x_ref, w3_ref, b3_ref, w4_ref, b4_ref,
                  o_ref):
    vm = vm_ref[...]
    selp = selp_ref[...]
    deficit = selp[:, K:K + 1]

    jg = jax.lax.broadcasted_iota(jnp.int32, (KP, TILEA), 0)
    cg = jax.lax.broadcasted_iota(jnp.int32, (KP, TILEA), 1)
    G = jnp.where(jg < K, (jg == cg % K).astype(jnp.bfloat16), 0)
    W = jnp.dot(selp.astype(jnp.bfloat16), G,
                preferred_element_type=jnp.float32)
    ri = jax.lax.broadcasted_iota(jnp.int32, (TILEA, TILEA), 0)
    ci = jax.lax.broadcasted_iota(jnp.int32, (TILEA, TILEA), 1)
    W = jnp.where((ri // K) == (ci // K), W, 0.0)
    W = W + jnp.where(ri == ci, deficit, 0.0)

    agg = jnp.dot(W.astype(jnp.bfloat16), vm,
                  preferred_element_type=jnp.float32)
    norm = jnp.sqrt(jnp.sum(agg * agg, axis=1, keepdims=True)) + 1e-8
    x2 = (x_ref[...] + agg / norm).astype(jnp.bfloat16)

    hm = jnp.dot(x2, w3_ref[...], preferred_element_type=jnp.float32)
    hm = jnp.maximum(hm + b3_ref[...], 0.0).astype(jnp.bfloat16)
    emb = jnp.dot(hm, w4_ref[...], preferred_element_type=jnp.float32)
    emb = emb + b4_ref[...]
    norm2 = jnp.sqrt(jnp.sum(emb * emb, axis=1, keepdims=True)) + 1e-8
    o_ref[...] = emb / norm2


def _prep_rt_body(r_ref, rt_ref):
    rt_ref[...] = jnp.concatenate(
        [r_ref[...].T, jnp.zeros((KP - K, r_ref.shape[0]), jnp.float32)],
        axis=0)


def _prep_rt(R):
    return pl.pallas_call(
        _prep_rt_body,
        grid=(M // 512,),
        in_specs=[pl.BlockSpec((512, K), lambda i: (i, 0))],
        out_specs=pl.BlockSpec((KP, 512), lambda i: (0, i)),
        out_shape=jax.ShapeDtypeStruct((KP, M), jnp.float32),
        compiler_params=_PARALLEL,
    )(R)


def _post_body(selt_ref, selp_ref):
    selp_ref[...] = selt_ref[...].T


def _post(selt):
    return pl.pallas_call(
        _post_body,
        grid=(M // 512,),
        in_specs=[pl.BlockSpec((KP, 512), lambda i: (0, i))],
        out_specs=pl.BlockSpec((512, KP), lambda i: (i, 0)),
        out_shape=jax.ShapeDtypeStruct((M, KP), jnp.float32),
        compiler_params=_PARALLEL,
    )(selt)


def _prep_body(gw1_ref, nw1_ref, nw2_ref, mw1_ref, mw2_ref, gw2_ref,
               w1ig_ref, w1in_ref, w1eg_ref, w1en_ref, w2n_ref,
               w3_ref, w4_ref, gw2c_ref):
    w1ig_ref[...] = gw1_ref[:, :D].astype(jnp.bfloat16).T
    w1in_ref[...] = nw1_ref[:, :D].astype(jnp.bfloat16).T
    pad = jnp.zeros((3, gw1_ref.shape[0]), jnp.bfloat16)
    w1eg_ref[...] = jnp.concatenate(
        [gw1_ref[:, D:].astype(jnp.bfloat16).T, pad], axis=0)
    w1en_ref[...] = jnp.concatenate(
        [nw1_ref[:, D:].astype(jnp.bfloat16).T, pad], axis=0)
    w2n_ref[...] = nw2_ref[...].astype(jnp.bfloat16).T
    w3_ref[...] = mw1_ref[...].astype(jnp.bfloat16).T
    w4_ref[...] = mw2_ref[...].astype(jnp.bfloat16).T
    gw2c = gw2_ref[...].astype(jnp.bfloat16).T
    gw2c_ref[...] = jnp.concatenate(
        [gw2c, jnp.zeros((gw2c.shape[0], 127), jnp.bfloat16)], axis=1)


def _prep_weights(gw1, nw1, nw2, mw1, mw2, gw2):
    g = 8
    r1 = D // g
    r2 = E // g
    return pl.pallas_call(
        _prep_body,
        grid=(g,),
        in_specs=[pl.BlockSpec((r1, D + 5), lambda i: (i, 0)),
                  pl.BlockSpec((r1, D + 5), lambda i: (i, 0)),
                  pl.BlockSpec((r1, D), lambda i: (i, 0)),
                  pl.BlockSpec((r1, D), lambda i: (i, 0)),
                  pl.BlockSpec((r2, D), lambda i: (i, 0)),
                  pl.BlockSpec((1, r1), lambda i: (0, i))],
        out_specs=[pl.BlockSpec((D, r1), lambda i: (0, i)),
                   pl.BlockSpec((D, r1), lambda i: (0, i)),
                   pl.BlockSpec((8, r1), lambda i: (0, i)),
                   pl.BlockSpec((8, r1), lambda i: (0, i)),
                   pl.BlockSpec((D, r1), lambda i: (0, i)),
                   pl.BlockSpec((D, r1), lambda i: (0, i)),
                   pl.BlockSpec((D, r2), lambda i: (0, i)),
                   pl.BlockSpec((r1, 128), lambda i: (i, 0))],
        out_shape=[jax.ShapeDtypeStruct((D, D), jnp.bfloat16),
                   jax.ShapeDtypeStruct((D, D), jnp.bfloat16),
                   jax.ShapeDtypeStruct((8, D), jnp.bfloat16),
                   jax.ShapeDtypeStruct((8, D), jnp.bfloat16),
                   jax.ShapeDtypeStruct((D, D), jnp.bfloat16),
                   jax.ShapeDtypeStruct((D, D), jnp.bfloat16),
                   jax.ShapeDtypeStruct((D, E), jnp.bfloat16),
                   jax.ShapeDtypeStruct((D, 128), jnp.bfloat16)],
        compiler_params=_PARALLEL,
    )(gw1, nw1, nw2, mw1, mw2, gw2)


def _row_spec(n, t=TILE):
    return pl.BlockSpec((t, n), lambda i: (i, 0))


def _full_spec(m, n):
    return pl.BlockSpec((m, n), lambda i: (0, 0))


@jax.jit
def _run(images, bboxes, img_range, gw1, gb1, gw2, gb2, nw1, nb1, nw2, nb2,
         mw1, mb1, mw2, mb2):
    X = images.reshape(M, D)
    bb = bboxes.reshape(M, 4)
    R = img_range.reshape(M, K)

    rt = _prep_rt(R)
    (w1ig, w1in, w1eg, w1en, w2n, w3, w4, gw2c) = _prep_weights(
        gw1, nw1, nw2, mw1, mw2, gw2)
    gb1r = gb1[None, :]
    nb1r = nb1[None, :]
    nb2r = nb2[None, :]
    gb2r = gb2[None, :]
    b3 = mb1[None, :]
    b4 = mb2[None, :]

    selp = _post(_select(rt))

    vm = pl.pallas_call(
        _kernelA_body,
        grid=(M // TILE,),
        in_specs=[_row_spec(D), _row_spec(4), _full_spec(D, D),
                  _full_spec(D, D), _full_spec(8, D), _full_spec(8, D),
                  _full_spec(1, D), _full_spec(1, D), _full_spec(D, D),
                  _full_spec(1, D), _full_spec(D, 128), _full_spec(1, 1)],
        out_specs=_row_spec(D),
        out_shape=jax.ShapeDtypeStruct((M, D), jnp.bfloat16),
        compiler_params=_PARALLEL,
    )(X, bb, w1ig, w1in, w1eg, w1en, gb1r, nb1r, w2n, nb2r, gw2c, gb2r)

    emb = pl.pallas_call(
        _kernelB_body,
        grid=(M // TILEA,),
        in_specs=[pl.BlockSpec((TILEA, KP), lambda i: (i, 0)),
                  _row_spec(D, TILEA), _row_spec(D, TILEA),
                  _full_spec(D, D), _full_spec(1, D),
                  _full_spec(D, E), _full_spec(1, E)],
        out_specs=_row_spec(E, TILEA),
        out_shape=jax.ShapeDtypeStruct((M, E), jnp.float32),
        compiler_params=_PARALLEL,
    )(selp, vm, X, w3, b3, w4, b4)

    return emb.reshape(B, K, E)


def kernel(images, bboxes, img_range, gw1, gb1, gw2, gb2, nw1, nb1, nw2, nb2,
           mw1, mb1, mw2, mb2):
    return _run(images, bboxes, img_range, gw1, gb1, gw2, gb2, nw1, nb1,
                nw2, nb2, mw1, mb1, mw2, mb2)
